# Initial kernel scaffold; baseline (speedup 1.0000x reference)
import jax
import jax.numpy as jnp
from jax.experimental import pallas as pl


def kernel(position_ids, pe):
    raise NotImplementedError("write your pallas kernel here")


# speedup vs baseline: 2.0990x; 2.0990x over previous
import functools

import jax
import jax.numpy as jnp
from jax import lax
from jax.experimental import pallas as pl
from jax.experimental.pallas import tpu as pltpu
from jax.experimental.pallas import tpu_sc as plsc

_B, _S, _D, _V = 4, 8192, 1024, 8192
_NB = _B * _S
_NC, _NS = 2, 16
_NW = _NC * _NS
_BPW = _NB // _NW
_C = 16
_NCHUNK = _BPW // _C

_mesh = plsc.VectorSubcoreMesh(core_axis_name="c", subcore_axis_name="s")


@functools.partial(
    pl.kernel,
    mesh=_mesh,
    out_type=jax.ShapeDtypeStruct((_NB, _D), jnp.float32),
    scratch_types=[
        pltpu.VMEM((_NCHUNK, _C), jnp.int32),
        pltpu.VMEM((_C, _D), jnp.float32),
        pltpu.VMEM((_C, _D), jnp.float32),
        pltpu.SemaphoreType.DMA,
        pltpu.SemaphoreType.DMA,
        pltpu.SemaphoreType.DMA,
        pltpu.SemaphoreType.DMA,
    ],
)
def _gather(idx_hbm, table_hbm, out_hbm, idx_v, buf0, buf1,
            g0sem, g1sem, s0sem, s1sem):
    wid = lax.axis_index("s") * _NC + lax.axis_index("c")
    base = wid * _BPW
    pltpu.sync_copy(idx_hbm.at[wid], idx_v)

    def step(j, carry):
        c0 = j * 2
        c1 = c0 + 1
        cp_g0 = pltpu.make_async_copy(table_hbm.at[idx_v.at[c0]], buf0, g0sem)
        cp_g1 = pltpu.make_async_copy(table_hbm.at[idx_v.at[c1]], buf1, g1sem)
        cp_g0.start()
        cp_g1.start()
        cp_g0.wait()
        cp_s0 = pltpu.make_async_copy(
            buf0, out_hbm.at[pl.ds(base + c0 * _C, _C)], s0sem)
        cp_s0.start()
        cp_g1.wait()
        cp_s1 = pltpu.make_async_copy(
            buf1, out_hbm.at[pl.ds(base + c1 * _C, _C)], s1sem)
        cp_s1.start()
        cp_s0.wait()
        cp_s1.wait()
        return carry

    lax.fori_loop(0, _NCHUNK // 2, step, 0)


def kernel(position_ids, pe):
    idx = position_ids.astype(jnp.int32).reshape(_NW, _NCHUNK, _C)
    out = _gather(idx, pe)
    return out.reshape(_B, _S, _D)


# speedup vs baseline: 2.3607x; 1.1247x over previous
import functools

import jax
import jax.numpy as jnp
from jax import lax
from jax.experimental import pallas as pl
from jax.experimental.pallas import tpu as pltpu
from jax.experimental.pallas import tpu_sc as plsc

_B, _S, _D, _V = 4, 8192, 1024, 8192
_NB = _B * _S
_NC, _NS = 2, 16
_NW = _NC * _NS
_BPW = _NB // _NW
_C = 16
_NCHUNK = _BPW // _C
_NSLOT = 4

_mesh = plsc.VectorSubcoreMesh(core_axis_name="c", subcore_axis_name="s")


@functools.partial(
    pl.kernel,
    mesh=_mesh,
    out_type=jax.ShapeDtypeStruct((_NB, _D), jnp.float32),
    scratch_types=[
        pltpu.VMEM((_NCHUNK, _C), jnp.int32),
        pltpu.VMEM((_C, _D), jnp.float32),
        pltpu.VMEM((_C, _D), jnp.float32),
        pltpu.VMEM((_C, _D), jnp.float32),
        pltpu.VMEM((_C, _D), jnp.float32),
        pltpu.SemaphoreType.DMA,
        pltpu.SemaphoreType.DMA,
        pltpu.SemaphoreType.DMA,
        pltpu.SemaphoreType.DMA,
        pltpu.SemaphoreType.DMA,
        pltpu.SemaphoreType.DMA,
        pltpu.SemaphoreType.DMA,
        pltpu.SemaphoreType.DMA,
    ],
)
def _gather(idx_hbm, table_hbm, out_hbm, idx_v,
            buf0, buf1, buf2, buf3,
            g0, g1, g2, g3, s0, s1, s2, s3):
    wid = lax.axis_index("s") * _NC + lax.axis_index("c")
    base = wid * _BPW
    pltpu.sync_copy(idx_hbm.at[wid], idx_v)

    bufs = (buf0, buf1, buf2, buf3)
    gsem = (g0, g1, g2, g3)
    ssem = (s0, s1, s2, s3)

    def gcp(ch, slot):
        return pltpu.make_async_copy(
            table_hbm.at[idx_v.at[ch]], bufs[slot], gsem[slot])

    def scp(ch, slot):
        return pltpu.make_async_copy(
            bufs[slot], out_hbm.at[pl.ds(base + ch * _C, _C)], ssem[slot])

    gcp(0, 0).start()
    gcp(1, 1).start()
    gcp(2, 2).start()
    gcp(0, 0).wait()
    scp(0, 0).start()
    gcp(3, 3).start()
    gcp(1, 1).wait()
    scp(1, 1).start()
    scp(0, 0).wait()
    gcp(4, 0).start()
    gcp(2, 2).wait()
    scp(2, 2).start()
    scp(1, 1).wait()
    gcp(5, 1).start()
    gcp(3, 3).wait()
    scp(3, 3).start()

    def body(j, carry):
        h0 = j * _NSLOT
        for u in range(_NSLOT):
            h = h0 + u
            b = u
            b2 = (u + 2) % _NSLOT
            scp(h - 2, b2).wait()
            gcp(h + 2, b2).start()
            gcp(h, b).wait()
            scp(h, b).start()
        return carry

    lax.fori_loop(1, _NCHUNK // _NSLOT - 1, body, 0)

    scp(58, 2).wait()
    gcp(62, 2).start()
    gcp(60, 0).wait()
    scp(60, 0).start()
    scp(59, 3).wait()
    gcp(63, 3).start()
    gcp(61, 1).wait()
    scp(61, 1).start()
    scp(60, 0).wait()
    gcp(62, 2).wait()
    scp(62, 2).start()
    scp(61, 1).wait()
    gcp(63, 3).wait()
    scp(63, 3).start()
    scp(62, 2).wait()
    scp(63, 3).wait()


def kernel(position_ids, pe):
    idx = position_ids.astype(jnp.int32).reshape(_NW, _NCHUNK, _C)
    out = _gather(idx, pe)
    return out.reshape(_B, _S, _D)
